# baseline (speedup 1.0000x reference)
import functools

import jax
import jax.numpy as jnp
from jax import lax
from jax.experimental import pallas as pl
from jax.experimental.pallas import tpu as pltpu
from jax.experimental.pallas import tpu_sc as plsc

N = 100000
F = 128
G = 64
EPS = 1e-06

C = 128
NFULL = N // C
TAIL = N - NFULL * C
NW = 32
CPW = (NFULL + NW - 1) // NW

R = 2000
NB = N // R


def _sc_body(x_hbm, b_hbm, seg_out, xbuf, idx_v, idx_tail, zseg, seg_sh):
    cid = lax.axis_index("c")
    sid = lax.axis_index("s")
    w = sid * 2 + cid

    def _z(g, _):
        for l in range(F // 16):
            zseg[g, pl.ds(16 * l, 16)] = jnp.zeros((16,), jnp.float32)
        return _
    lax.fori_loop(0, 4, _z, 0)
    pltpu.sync_copy(zseg, seg_sh.at[pl.ds(sid * 4, 4), :])
    plsc.subcore_barrier()

    def _square_rows(nrows):
        def _sq(r, _):
            for l in range(F // 16):
                v = xbuf[r, pl.ds(16 * l, 16)]
                xbuf[r, pl.ds(16 * l, 16)] = v * v
            return _
        lax.fori_loop(0, nrows, _sq, 0)

    def _chunk(j, _):
        ck = w + NW * j

        @pl.when(ck < NFULL)
        def _do():
            base = ck * C
            pltpu.sync_copy(b_hbm.at[pl.ds(base, C)], idx_v)
            pltpu.sync_copy(x_hbm.at[pl.ds(base, C), :], xbuf)
            _square_rows(C)
            pltpu.sync_copy(xbuf, seg_sh.at[idx_v], add=True)
        return _
    lax.fori_loop(0, CPW, _chunk, 0)

    @pl.when(w == NW - 1)
    def _tail():
        base = NFULL * C
        pltpu.sync_copy(b_hbm.at[pl.ds(base, TAIL)], idx_tail)
        pltpu.sync_copy(x_hbm.at[pl.ds(base, TAIL), :],
                        xbuf.at[pl.ds(0, TAIL), :])
        _square_rows(TAIL)
        pltpu.sync_copy(xbuf.at[pl.ds(0, TAIL), :],
                        seg_sh.at[idx_tail], add=True)

    plsc.subcore_barrier()

    @pl.when(sid == 0)
    def _emit():
        pltpu.sync_copy(seg_sh, seg_out.at[cid])


@functools.partial(
    pl.kernel,
    out_type=jax.ShapeDtypeStruct((2, G, F), jnp.float32),
    mesh=plsc.VectorSubcoreMesh(core_axis_name="c", subcore_axis_name="s"),
    scratch_types=[
        pltpu.VMEM((C, F), jnp.float32),
        pltpu.VMEM((C,), jnp.int32),
        pltpu.VMEM((TAIL,), jnp.int32),
        pltpu.VMEM((4, F), jnp.float32),
        pltpu.VMEM_SHARED((G, F), jnp.float32),
    ],
)
def _sc_segsum(x_hbm, b_hbm, seg_out, *scratch):
    _sc_body(x_hbm, b_hbm, seg_out, *scratch)


def _cnt_body(b_ref, out_ref):
    def _step(k, acc):
        b = b_ref[k, 0, :]
        iota_g = jax.lax.broadcasted_iota(jnp.int32, (G, R), 0)
        onehot = (iota_g == b[None, :]).astype(jnp.float32)
        return acc + jnp.sum(onehot, axis=1, keepdims=True)
    cnt = lax.fori_loop(0, NB, _step, jnp.zeros((G, 1), jnp.float32))
    out_ref[...] = jnp.broadcast_to(cnt, (G, F))


def _tc_body(x_ref, b_ref, seg_ref, cnt_ref, w_ref, bias_ref, out_ref,
             inv_ref):
    i = pl.program_id(0)

    @pl.when(i == 0)
    def _mk_inv():
        seg = seg_ref[0] + seg_ref[1]
        mean_sq = seg / jnp.maximum(cnt_ref[...], 1.0)
        inv_ref[...] = jax.lax.rsqrt(mean_sq + EPS)

    b = b_ref[0, 0, :]
    iota_g = jax.lax.broadcasted_iota(jnp.int32, (R, G), 1)
    onehot = (iota_g == b[:, None]).astype(jnp.float32)
    inv = jnp.dot(onehot, inv_ref[...],
                  preferred_element_type=jnp.float32)
    out_ref[...] = x_ref[...] * inv * w_ref[0, :] + bias_ref[0, :]


def kernel(x, batch, weight, bias):
    b32 = batch.astype(jnp.int32)
    b3 = b32.reshape(NB, 1, R)
    seg = _sc_segsum(x, b32)
    cnt = pl.pallas_call(
        _cnt_body,
        grid=(1,),
        in_specs=[pl.BlockSpec((NB, 1, R), lambda i: (0, 0, 0))],
        out_specs=pl.BlockSpec((G, F), lambda i: (0, 0)),
        out_shape=jax.ShapeDtypeStruct((G, F), jnp.float32),
    )(b3)
    w2 = weight.reshape(1, F)
    bias2 = bias.reshape(1, F)
    return pl.pallas_call(
        _tc_body,
        grid=(NB,),
        in_specs=[
            pl.BlockSpec((R, F), lambda i: (i, 0)),
            pl.BlockSpec((1, 1, R), lambda i: (i, 0, 0)),
            pl.BlockSpec((2, G, F), lambda i: (0, 0, 0)),
            pl.BlockSpec((G, F), lambda i: (0, 0)),
            pl.BlockSpec((1, F), lambda i: (0, 0)),
            pl.BlockSpec((1, F), lambda i: (0, 0)),
        ],
        out_specs=pl.BlockSpec((R, F), lambda i: (i, 0)),
        out_shape=jax.ShapeDtypeStruct((N, F), jnp.float32),
        scratch_shapes=[pltpu.VMEM((G, F), jnp.float32)],
    )(x, b3, seg, cnt, w2, bias2)


# speedup vs baseline: 1.1216x; 1.1216x over previous
import functools

import jax
import jax.numpy as jnp
from jax import lax
from jax.experimental import pallas as pl
from jax.experimental.pallas import tpu as pltpu
from jax.experimental.pallas import tpu_sc as plsc

N = 100000
F = 128
G = 64
EPS = 1e-06

C = 128
NFULL = N // C
TAIL = N - NFULL * C
NW = 32
CPW = (NFULL + NW - 1) // NW

R = 4000
NB = N // R


def _sc_body(x_hbm, b_hbm, seg_out, xbuf, idx_v, idx_tail, zseg, seg_sh):
    cid = lax.axis_index("c")
    sid = lax.axis_index("s")
    w = sid * 2 + cid

    def _z(g, _):
        for l in range(F // 16):
            zseg[g, pl.ds(16 * l, 16)] = jnp.zeros((16,), jnp.float32)
        return _
    lax.fori_loop(0, 4, _z, 0)
    pltpu.sync_copy(zseg, seg_sh.at[pl.ds(sid * 4, 4), :])
    plsc.subcore_barrier()

    def _square_rows(nrows):
        def _sq(r, _):
            for l in range(F // 16):
                v = xbuf[r, pl.ds(16 * l, 16)]
                xbuf[r, pl.ds(16 * l, 16)] = v * v
            return _
        lax.fori_loop(0, nrows, _sq, 0)

    def _chunk(j, _):
        ck = w + NW * j

        @pl.when(ck < NFULL)
        def _do():
            base = ck * C
            pltpu.sync_copy(b_hbm.at[pl.ds(base, C)], idx_v)
            pltpu.sync_copy(x_hbm.at[pl.ds(base, C), :], xbuf)
            _square_rows(C)
            pltpu.sync_copy(xbuf, seg_sh.at[idx_v], add=True)
        return _
    lax.fori_loop(0, CPW, _chunk, 0)

    @pl.when(w == NW - 1)
    def _tail():
        base = NFULL * C
        pltpu.sync_copy(b_hbm.at[pl.ds(base, TAIL)], idx_tail)
        pltpu.sync_copy(x_hbm.at[pl.ds(base, TAIL), :],
                        xbuf.at[pl.ds(0, TAIL), :])
        _square_rows(TAIL)
        pltpu.sync_copy(xbuf.at[pl.ds(0, TAIL), :],
                        seg_sh.at[idx_tail], add=True)

    plsc.subcore_barrier()

    @pl.when(sid == 0)
    def _emit():
        pltpu.sync_copy(seg_sh, seg_out.at[cid])


@functools.partial(
    pl.kernel,
    out_type=jax.ShapeDtypeStruct((2, G, F), jnp.float32),
    mesh=plsc.VectorSubcoreMesh(core_axis_name="c", subcore_axis_name="s"),
    scratch_types=[
        pltpu.VMEM((C, F), jnp.float32),
        pltpu.VMEM((C,), jnp.int32),
        pltpu.VMEM((TAIL,), jnp.int32),
        pltpu.VMEM((4, F), jnp.float32),
        pltpu.VMEM_SHARED((G, F), jnp.float32),
    ],
)
def _sc_segsum(x_hbm, b_hbm, seg_out, *scratch):
    _sc_body(x_hbm, b_hbm, seg_out, *scratch)


def _cnt_body(b_ref, out_ref):
    def _step(k, acc):
        b = b_ref[k, 0, :]
        iota_g = jax.lax.broadcasted_iota(jnp.int32, (G, R), 0)
        onehot = (iota_g == b[None, :]).astype(jnp.float32)
        return acc + jnp.sum(onehot, axis=1, keepdims=True)
    cnt = lax.fori_loop(0, NB, _step, jnp.zeros((G, 1), jnp.float32))
    out_ref[...] = jnp.broadcast_to(cnt, (G, F))


def _tc_body(x_ref, b_ref, seg_ref, cnt_ref, w_ref, bias_ref, out_ref,
             inv_ref):
    i = pl.program_id(0)

    @pl.when(i == 0)
    def _mk_inv():
        seg = seg_ref[0] + seg_ref[1]
        mean_sq = seg / jnp.maximum(cnt_ref[...], 1.0)
        inv_ref[...] = jax.lax.rsqrt(mean_sq + EPS)

    b = b_ref[0, 0, :]
    iota_g = jax.lax.broadcasted_iota(jnp.int32, (R, G), 1)
    onehot = (iota_g == b[:, None]).astype(jnp.float32)
    inv = jnp.dot(onehot, inv_ref[...],
                  preferred_element_type=jnp.float32)
    out_ref[...] = x_ref[...] * inv * w_ref[0, :] + bias_ref[0, :]


def kernel(x, batch, weight, bias):
    b32 = batch.astype(jnp.int32)
    b3 = b32.reshape(NB, 1, R)
    seg = _sc_segsum(x, b32)
    cnt = pl.pallas_call(
        _cnt_body,
        grid=(1,),
        in_specs=[pl.BlockSpec((NB, 1, R), lambda i: (0, 0, 0))],
        out_specs=pl.BlockSpec((G, F), lambda i: (0, 0)),
        out_shape=jax.ShapeDtypeStruct((G, F), jnp.float32),
    )(b3)
    w2 = weight.reshape(1, F)
    bias2 = bias.reshape(1, F)
    return pl.pallas_call(
        _tc_body,
        grid=(NB,),
        in_specs=[
            pl.BlockSpec((R, F), lambda i: (i, 0)),
            pl.BlockSpec((1, 1, R), lambda i: (i, 0, 0)),
            pl.BlockSpec((2, G, F), lambda i: (0, 0, 0)),
            pl.BlockSpec((G, F), lambda i: (0, 0)),
            pl.BlockSpec((1, F), lambda i: (0, 0)),
            pl.BlockSpec((1, F), lambda i: (0, 0)),
        ],
        out_specs=pl.BlockSpec((R, F), lambda i: (i, 0)),
        out_shape=jax.ShapeDtypeStruct((N, F), jnp.float32),
        scratch_shapes=[pltpu.VMEM((G, F), jnp.float32)],
    )(x, b3, seg, cnt, w2, bias2)


# speedup vs baseline: 1.5963x; 1.4233x over previous
import functools

import jax
import jax.numpy as jnp
from jax import lax
from jax.experimental import pallas as pl
from jax.experimental.pallas import tpu as pltpu
from jax.experimental.pallas import tpu_sc as plsc

N = 100000
F = 128
G = 64
EPS = 1e-06

C = 128
NFULL = N // C
TAIL = N - NFULL * C
NW = 32
CPW = (NFULL + NW - 1) // NW

R = 4000
NB = N // R


def _sc_body(x_hbm, b_hbm, seg_out, xb0, xb1, id0, id1, idx_tail, idq,
             acctab, seg_sh, sem0, sem1):
    cid = lax.axis_index("c")
    sid = lax.axis_index("s")
    w = sid * 2 + cid
    xb = (xb0, xb1)
    idx = (id0, id1)
    sem = (sem0, sem1)
    NL = F // 16

    def _z(g, _):
        for l in range(NL):
            acctab[g, pl.ds(16 * l, 16)] = jnp.zeros((16,), jnp.float32)
        return _
    lax.fori_loop(0, G, _z, 0)
    for k in range(G // 16):
        idq[pl.ds(16 * k, 16)] = lax.iota(jnp.int32, 16) + 16 * k
    pltpu.sync_copy(acctab.at[pl.ds(0, 4), :], seg_sh.at[pl.ds(sid * 4, 4), :])
    plsc.subcore_barrier()

    def _start(j, b):
        ck = w + NW * j

        @pl.when(ck < NFULL)
        def _():
            pltpu.async_copy(x_hbm.at[pl.ds(ck * C, C), :], xb[b], sem[b])
            pltpu.async_copy(b_hbm.at[pl.ds(ck * C, C)], idx[b], sem[b])

    def _wait(b):
        pltpu.make_async_copy(x_hbm.at[pl.ds(0, C), :], xb[b], sem[b]).wait()
        pltpu.make_async_copy(b_hbm.at[pl.ds(0, C)], idx[b], sem[b]).wait()

    def _process(b):
        v_first = idx[b][pl.ds(0, 16)]
        v_last = idx[b][pl.ds(C - 16, 16)]
        uni = v_first[0] == v_last[15]

        @pl.when(uni)
        def _uniform():
            def _acc(r, a):
                return tuple(a[l] + xb[b][r, pl.ds(16 * l, 16)] *
                             xb[b][r, pl.ds(16 * l, 16)] for l in range(NL))
            a = lax.fori_loop(0, C, _acc,
                              tuple(jnp.zeros((16,), jnp.float32)
                                    for _ in range(NL)))
            g = v_first[0]
            for l in range(NL):
                acctab[g, pl.ds(16 * l, 16)] += a[l]

        @pl.when(jnp.logical_not(uni))
        def _boundary():
            def _sq(r, _):
                for l in range(NL):
                    v = xb[b][r, pl.ds(16 * l, 16)]
                    xb[b][r, pl.ds(16 * l, 16)] = v * v
                return _
            lax.fori_loop(0, C, _sq, 0)
            pltpu.sync_copy(xb[b], seg_sh.at[idx[b]], add=True)

    _start(0, 0)
    T = (CPW + 1) // 2

    def _loop(t, _):
        j0 = 2 * t
        j1 = 2 * t + 1

        @pl.when(w + NW * j0 < NFULL)
        def _even():
            _wait(0)
            _start(j1, 1)
            _process(0)

        @pl.when(w + NW * j1 < NFULL)
        def _odd():
            _wait(1)
            _start(j1 + 1, 0)
            _process(1)
        return _
    lax.fori_loop(0, T, _loop, 0)

    pltpu.sync_copy(acctab, seg_sh.at[idq], add=True)

    @pl.when(w == NW - 1)
    def _tail():
        base = NFULL * C
        pltpu.sync_copy(b_hbm.at[pl.ds(base, TAIL)], idx_tail)
        pltpu.sync_copy(x_hbm.at[pl.ds(base, TAIL), :],
                        xb0.at[pl.ds(0, TAIL), :])

        def _sq(r, _):
            for l in range(NL):
                v = xb0[r, pl.ds(16 * l, 16)]
                xb0[r, pl.ds(16 * l, 16)] = v * v
            return _
        lax.fori_loop(0, TAIL, _sq, 0)
        pltpu.sync_copy(xb0.at[pl.ds(0, TAIL), :],
                        seg_sh.at[idx_tail], add=True)

    plsc.subcore_barrier()

    @pl.when(sid == 0)
    def _emit():
        pltpu.sync_copy(seg_sh, seg_out.at[cid])


@functools.partial(
    pl.kernel,
    out_type=jax.ShapeDtypeStruct((2, G, F), jnp.float32),
    mesh=plsc.VectorSubcoreMesh(core_axis_name="c", subcore_axis_name="s"),
    scratch_types=[
        pltpu.VMEM((C, F), jnp.float32),
        pltpu.VMEM((C, F), jnp.float32),
        pltpu.VMEM((C,), jnp.int32),
        pltpu.VMEM((C,), jnp.int32),
        pltpu.VMEM((TAIL,), jnp.int32),
        pltpu.VMEM((G,), jnp.int32),
        pltpu.VMEM((G, F), jnp.float32),
        pltpu.VMEM_SHARED((G, F), jnp.float32),
        pltpu.SemaphoreType.DMA,
        pltpu.SemaphoreType.DMA,
    ],
)
def _sc_segsum(x_hbm, b_hbm, seg_out, *scratch):
    _sc_body(x_hbm, b_hbm, seg_out, *scratch)


def _cnt_body(b_ref, out_ref):
    def _step(k, acc):
        b = b_ref[k, 0, :]
        iota_g = jax.lax.broadcasted_iota(jnp.int32, (G, R), 0)
        onehot = (iota_g == b[None, :]).astype(jnp.float32)
        return acc + jnp.sum(onehot, axis=1, keepdims=True)
    cnt = lax.fori_loop(0, NB, _step, jnp.zeros((G, 1), jnp.float32))
    out_ref[...] = jnp.broadcast_to(cnt, (G, F))


def _tc_body(x_ref, b_ref, seg_ref, cnt_ref, w_ref, bias_ref, out_ref,
             inv_ref):
    i = pl.program_id(0)

    @pl.when(i == 0)
    def _mk_inv():
        seg = seg_ref[0] + seg_ref[1]
        mean_sq = seg / jnp.maximum(cnt_ref[...], 1.0)
        inv_ref[...] = jax.lax.rsqrt(mean_sq + EPS)

    b = b_ref[0, 0, :]
    iota_g = jax.lax.broadcasted_iota(jnp.int32, (R, G), 1)
    onehot = (iota_g == b[:, None]).astype(jnp.float32)
    inv = jnp.dot(onehot, inv_ref[...],
                  preferred_element_type=jnp.float32)
    out_ref[...] = x_ref[...] * inv * w_ref[0, :] + bias_ref[0, :]


def kernel(x, batch, weight, bias):
    b32 = batch.astype(jnp.int32)
    b3 = b32.reshape(NB, 1, R)
    seg = _sc_segsum(x, b32)
    cnt = pl.pallas_call(
        _cnt_body,
        grid=(1,),
        in_specs=[pl.BlockSpec((NB, 1, R), lambda i: (0, 0, 0))],
        out_specs=pl.BlockSpec((G, F), lambda i: (0, 0)),
        out_shape=jax.ShapeDtypeStruct((G, F), jnp.float32),
    )(b3)
    w2 = weight.reshape(1, F)
    bias2 = bias.reshape(1, F)
    return pl.pallas_call(
        _tc_body,
        grid=(NB,),
        in_specs=[
            pl.BlockSpec((R, F), lambda i: (i, 0)),
            pl.BlockSpec((1, 1, R), lambda i: (i, 0, 0)),
            pl.BlockSpec((2, G, F), lambda i: (0, 0, 0)),
            pl.BlockSpec((G, F), lambda i: (0, 0)),
            pl.BlockSpec((1, F), lambda i: (0, 0)),
            pl.BlockSpec((1, F), lambda i: (0, 0)),
        ],
        out_specs=pl.BlockSpec((R, F), lambda i: (i, 0)),
        out_shape=jax.ShapeDtypeStruct((N, F), jnp.float32),
        scratch_shapes=[pltpu.VMEM((G, F), jnp.float32)],
    )(x, b3, seg, cnt, w2, bias2)


# speedup vs baseline: 1.8352x; 1.1496x over previous
import functools

import jax
import jax.numpy as jnp
from jax import lax
from jax.experimental import pallas as pl
from jax.experimental.pallas import tpu as pltpu
from jax.experimental.pallas import tpu_sc as plsc

N = 100000
F = 128
G = 64
EPS = 1e-06

C = 128
NFULL = N // C
TAIL = N - NFULL * C
NW = 32
EXTRA = NFULL % NW
BASE_CPW = NFULL // NW
NPAIR = (BASE_CPW + 2) // 2

R = 10000
NB = N // R


def _sc_body(x_hbm, b_hbm, seg_out, xba, xbb, ia0, ia1, ib0, ib1,
             idx_tail, idq, acctab, seg_sh, sem0, sem1):
    cid = lax.axis_index("c")
    sid = lax.axis_index("s")
    w = sid * 2 + cid
    xb = (xba, xbb)
    iA = (ia0, ia1)
    iB = (ib0, ib1)
    sem = (sem0, sem1)
    NL = F // 16

    start_w = BASE_CPW * w + jnp.minimum(w, EXTRA)
    end_w = start_w + BASE_CPW + jnp.where(w < EXTRA, 1, 0)

    def _z(g, _):
        for l in range(NL):
            acctab[g, pl.ds(16 * l, 16)] = jnp.zeros((16,), jnp.float32)
        return _
    lax.fori_loop(0, G, _z, 0)
    for k in range(G // 16):
        idq[pl.ds(16 * k, 16)] = lax.iota(jnp.int32, 16) + 16 * k
    pltpu.sync_copy(acctab.at[pl.ds(0, 4), :], seg_sh.at[pl.ds(sid * 4, 4), :])
    plsc.subcore_barrier()

    def _start(p, b):
        ck0 = start_w + 2 * p

        @pl.when(ck0 + 1 < end_w)
        def _full():
            pltpu.async_copy(x_hbm.at[pl.ds(ck0 * C, 2 * C), :], xb[b], sem[b])
            pltpu.async_copy(b_hbm.at[pl.ds(ck0 * C, C)], iA[b], sem[b])
            pltpu.async_copy(b_hbm.at[pl.ds(ck0 * C + C, C)], iB[b], sem[b])

        @pl.when(jnp.logical_and(ck0 < end_w, ck0 + 1 >= end_w))
        def _single():
            pltpu.async_copy(x_hbm.at[pl.ds(ck0 * C, C), :],
                             xb[b].at[pl.ds(0, C), :], sem[b])
            pltpu.async_copy(b_hbm.at[pl.ds(ck0 * C, C)], iA[b], sem[b])

    def _wait(p, b):
        ck0 = start_w + 2 * p

        @pl.when(ck0 + 1 < end_w)
        def _full():
            pltpu.make_async_copy(x_hbm.at[pl.ds(0, 2 * C), :], xb[b],
                                  sem[b]).wait()
            pltpu.make_async_copy(b_hbm.at[pl.ds(0, C)], iA[b], sem[b]).wait()
            pltpu.make_async_copy(b_hbm.at[pl.ds(0, C)], iB[b], sem[b]).wait()

        @pl.when(jnp.logical_and(ck0 < end_w, ck0 + 1 >= end_w))
        def _single():
            pltpu.make_async_copy(x_hbm.at[pl.ds(0, C), :],
                                  xb[b].at[pl.ds(0, C), :], sem[b]).wait()
            pltpu.make_async_copy(b_hbm.at[pl.ds(0, C)], iA[b], sem[b]).wait()

    def _accum(b, nrows, gvec):
        def _acc(r, a):
            return tuple(a[l] + xb[b][r, pl.ds(16 * l, 16)] *
                         xb[b][r, pl.ds(16 * l, 16)] for l in range(NL))
        a = lax.fori_loop(0, nrows, _acc,
                          tuple(jnp.zeros((16,), jnp.float32)
                                for _ in range(NL)))
        g = gvec[0]
        for l in range(NL):
            acctab[g, pl.ds(16 * l, 16)] += a[l]

    def _square(b, nrows):
        def _sq(r, _):
            for l in range(NL):
                v = xb[b][r, pl.ds(16 * l, 16)]
                xb[b][r, pl.ds(16 * l, 16)] = v * v
            return _
        lax.fori_loop(0, nrows, _sq, 0)

    def _process(p, b):
        ck0 = start_w + 2 * p

        @pl.when(ck0 + 1 < end_w)
        def _full():
            v_first = iA[b][pl.ds(0, 16)]
            v_last = iB[b][pl.ds(C - 16, 16)]
            uni = v_first[0] == v_last[15]

            @pl.when(uni)
            def _uniform():
                _accum(b, 2 * C, v_first)

            @pl.when(jnp.logical_not(uni))
            def _boundary():
                _square(b, 2 * C)
                pltpu.sync_copy(xb[b].at[pl.ds(0, C), :],
                                seg_sh.at[iA[b]], add=True)
                pltpu.sync_copy(xb[b].at[pl.ds(C, C), :],
                                seg_sh.at[iB[b]], add=True)

        @pl.when(jnp.logical_and(ck0 < end_w, ck0 + 1 >= end_w))
        def _single():
            v_first = iA[b][pl.ds(0, 16)]
            v_last = iA[b][pl.ds(C - 16, 16)]
            uni = v_first[0] == v_last[15]

            @pl.when(uni)
            def _uniform():
                _accum(b, C, v_first)

            @pl.when(jnp.logical_not(uni))
            def _boundary():
                _square(b, C)
                pltpu.sync_copy(xb[b].at[pl.ds(0, C), :],
                                seg_sh.at[iA[b]], add=True)

    _start(0, 0)
    T = (NPAIR + 1) // 2

    def _loop(t, _):
        p0 = 2 * t
        p1 = 2 * t + 1

        @pl.when(start_w + 2 * p0 < end_w)
        def _even():
            _start(p1, 1)
            _wait(p0, 0)
            _process(p0, 0)

        @pl.when(start_w + 2 * p1 < end_w)
        def _odd():
            _start(p1 + 1, 0)
            _wait(p1, 1)
            _process(p1, 1)
        return _
    lax.fori_loop(0, T, _loop, 0)

    pltpu.sync_copy(acctab, seg_sh.at[idq], add=True)

    @pl.when(w == NW - 1)
    def _tail():
        base = NFULL * C
        pltpu.sync_copy(b_hbm.at[pl.ds(base, TAIL)], idx_tail)
        pltpu.sync_copy(x_hbm.at[pl.ds(base, TAIL), :],
                        xba.at[pl.ds(0, TAIL), :])
        def _sq(r, _):
            for l in range(F // 16):
                v = xba[r, pl.ds(16 * l, 16)]
                xba[r, pl.ds(16 * l, 16)] = v * v
            return _
        lax.fori_loop(0, TAIL, _sq, 0)
        pltpu.sync_copy(xba.at[pl.ds(0, TAIL), :],
                        seg_sh.at[idx_tail], add=True)

    plsc.subcore_barrier()

    @pl.when(sid == 0)
    def _emit():
        pltpu.sync_copy(seg_sh, seg_out.at[cid])


@functools.partial(
    pl.kernel,
    out_type=jax.ShapeDtypeStruct((2, G, F), jnp.float32),
    mesh=plsc.VectorSubcoreMesh(core_axis_name="c", subcore_axis_name="s"),
    scratch_types=[
        pltpu.VMEM((2 * C, F), jnp.float32),
        pltpu.VMEM((2 * C, F), jnp.float32),
        pltpu.VMEM((C,), jnp.int32),
        pltpu.VMEM((C,), jnp.int32),
        pltpu.VMEM((C,), jnp.int32),
        pltpu.VMEM((C,), jnp.int32),
        pltpu.VMEM((TAIL,), jnp.int32),
        pltpu.VMEM((G,), jnp.int32),
        pltpu.VMEM((G, F), jnp.float32),
        pltpu.VMEM_SHARED((G, F), jnp.float32),
        pltpu.SemaphoreType.DMA,
        pltpu.SemaphoreType.DMA,
    ],
)
def _sc_segsum(x_hbm, b_hbm, seg_out, *scratch):
    _sc_body(x_hbm, b_hbm, seg_out, *scratch)


def _cnt_body(b_ref, out_ref):
    def _step(k, acc):
        b = b_ref[k, 0, :]
        iota_g = jax.lax.broadcasted_iota(jnp.int32, (G, R), 0)
        onehot = (iota_g == b[None, :]).astype(jnp.float32)
        return acc + jnp.sum(onehot, axis=1, keepdims=True)
    cnt = lax.fori_loop(0, NB, _step, jnp.zeros((G, 1), jnp.float32))
    out_ref[...] = jnp.broadcast_to(cnt, (G, F))


def _tc_body(x_ref, b_ref, seg_ref, cnt_ref, w_ref, bias_ref, out_ref,
             inv_ref):
    i = pl.program_id(0)

    @pl.when(i == 0)
    def _mk_inv():
        seg = seg_ref[0] + seg_ref[1]
        mean_sq = seg / jnp.maximum(cnt_ref[...], 1.0)
        inv_ref[...] = jax.lax.rsqrt(mean_sq + EPS)

    b = b_ref[0, 0, :]
    iota_g = jax.lax.broadcasted_iota(jnp.int32, (R, G), 1)
    onehot = (iota_g == b[:, None]).astype(jnp.float32)
    inv = jnp.dot(onehot, inv_ref[...],
                  preferred_element_type=jnp.float32)
    out_ref[...] = x_ref[...] * inv * w_ref[0, :] + bias_ref[0, :]


def kernel(x, batch, weight, bias):
    b32 = batch.astype(jnp.int32)
    b3 = b32.reshape(NB, 1, R)
    seg = _sc_segsum(x, b32)
    cnt = pl.pallas_call(
        _cnt_body,
        grid=(1,),
        in_specs=[pl.BlockSpec((NB, 1, R), lambda i: (0, 0, 0))],
        out_specs=pl.BlockSpec((G, F), lambda i: (0, 0)),
        out_shape=jax.ShapeDtypeStruct((G, F), jnp.float32),
    )(b3)
    w2 = weight.reshape(1, F)
    bias2 = bias.reshape(1, F)
    return pl.pallas_call(
        _tc_body,
        grid=(NB,),
        in_specs=[
            pl.BlockSpec((R, F), lambda i: (i, 0)),
            pl.BlockSpec((1, 1, R), lambda i: (i, 0, 0)),
            pl.BlockSpec((2, G, F), lambda i: (0, 0, 0)),
            pl.BlockSpec((G, F), lambda i: (0, 0)),
            pl.BlockSpec((1, F), lambda i: (0, 0)),
            pl.BlockSpec((1, F), lambda i: (0, 0)),
        ],
        out_specs=pl.BlockSpec((R, F), lambda i: (i, 0)),
        out_shape=jax.ShapeDtypeStruct((N, F), jnp.float32),
        scratch_shapes=[pltpu.VMEM((G, F), jnp.float32)],
    )(x, b3, seg, cnt, w2, bias2)
